# baseline (speedup 1.0000x reference)
import functools

import jax
import jax.numpy as jnp
from jax import lax
from jax.experimental import pallas as pl
from jax.experimental.pallas import tpu as pltpu
from jax.experimental.pallas import tpu_sc as plsc

NC = 2
NS = 16
NW = NC * NS
SB = 128

DIM = 64
K = 16
SUBC = 32


def _mesh():
    return plsc.VectorSubcoreMesh(core_axis_name="c", subcore_axis_name="s")


def _wid():
    return lax.axis_index("s") * NC + lax.axis_index("c")


def _sc_params():
    return pltpu.CompilerParams(use_tc_tiling_on_sc=False,
                                needs_layout_passes=False)


def _sc_params_tiled():
    return pltpu.CompilerParams(use_tc_tiling_on_sc=True,
                                needs_layout_passes=False)


_DNUMS = lax.GatherDimensionNumbers(
    offset_dims=(), collapsed_slice_dims=(0,), start_index_map=(0,))


def _lane_bcast(vec, m):
    return lax.gather(vec, jnp.full((16, 1), m, jnp.int32), _DNUMS, (1,),
                      mode=lax.GatherScatterMode.PROMISE_IN_BOUNDS)


def _make_gather_hop0(B):
    assert B == NW * SB

    @functools.partial(
        pl.kernel,
        out_type=[
            jax.ShapeDtypeStruct((B, DIM), jnp.float32),
            jax.ShapeDtypeStruct((B * K // 128, 128), jnp.int32),
            jax.ShapeDtypeStruct((B * K // 128, 128), jnp.int32),
        ],
        mesh=_mesh(),
        compiler_params=_sc_params_tiled(),
        scratch_types=[
            pltpu.VMEM((1, SB), jnp.int32),
            pltpu.VMEM((1, SB), jnp.int32),
            pltpu.VMEM((1, SB), jnp.int32),
            pltpu.VMEM((1, SB), jnp.int32),
            pltpu.VMEM((SB, 128), jnp.float32),
            pltpu.VMEM((SB, 128), jnp.int32),
            pltpu.VMEM((SB, 128), jnp.int32),
            pltpu.VMEM((SB, DIM), jnp.float32),
            pltpu.VMEM((SB * K // 128, 128), jnp.int32),
            pltpu.VMEM((SB * K // 128, 128), jnp.int32),
            pltpu.SemaphoreType.DMA,
        ],
    )
    def kern(u_hbm, v_hbm, usrp_hbm, aep_hbm, arp_hbm,
             ue_out, ne1_out, nr1_out,
             ui_v, vi_v, ui2_v, vi8_v, st_u, st_a, st_r,
             ue_v, ne1_v, nr1_v, sem):
        sb0 = _wid()
        pltpu.sync_copy(u_hbm.at[pl.ds(sb0, 1)], ui_v)
        pltpu.sync_copy(v_hbm.at[pl.ds(sb0, 1)], vi_v)

        def prep(lg, carry):
            col = pl.multiple_of(lg * 16, 16)
            ui2_v[0, pl.ds(col, 16)] = jnp.right_shift(ui_v[0, pl.ds(col, 16)], 1)
            vi8_v[0, pl.ds(col, 16)] = jnp.right_shift(vi_v[0, pl.ds(col, 16)], 3)
            return carry

        lax.fori_loop(0, 8, prep, 0)
        cs = [
            pltpu.async_copy(usrp_hbm.at[ui2_v.at[0]], st_u, sem),
            pltpu.async_copy(aep_hbm.at[vi8_v.at[0]], st_a, sem),
            pltpu.async_copy(arp_hbm.at[vi8_v.at[0]], st_r, sem),
        ]
        for c in cs:
            c.wait()
        iota16 = lax.iota(jnp.int32, 16)

        def extract(lg, carry):
            col = pl.multiple_of(lg * 16, 16)
            uu = ui_v[0, pl.ds(col, 16)]
            vv = vi_v[0, pl.ds(col, 16)]
            umod = jnp.bitwise_and(uu, 1)
            vmod = jnp.bitwise_and(vv, 7)
            for m in range(16):
                mrow = lg * 16 + m
                prow = 2 * lg + m // 8
                pcol = (m % 8) * K
                rowv = jnp.zeros((16,), jnp.int32) + mrow
                vcol = _lane_bcast(vmod, m) * 16 + iota16
                ne1_v[prow, pl.ds(pcol, 16)] = plsc.load_gather(st_a, [rowv, vcol])
                nr1_v[prow, pl.ds(pcol, 16)] = plsc.load_gather(st_r, [rowv, vcol])
                ubase = _lane_bcast(umod, m) * DIM + iota16
                for dd in range(DIM // 16):
                    ue_v[mrow, pl.ds(dd * 16, 16)] = plsc.load_gather(
                        st_u, [rowv, ubase + dd * 16])
            return carry

        lax.fori_loop(0, 8, extract, 0)
        row0 = sb0 * SB
        pltpu.sync_copy(ue_v, ue_out.at[pl.ds(row0, SB)])
        pr = SB * K // 128
        pltpu.sync_copy(ne1_v, ne1_out.at[pl.ds(sb0 * pr, pr)])
        pltpu.sync_copy(nr1_v, nr1_out.at[pl.ds(sb0 * pr, pr)])

    return kern


def _make_gather_hop1(n_idx, group):
    total_sb = n_idx // SB
    sb_per_w = total_sb // NW
    g = min(group, sb_per_w)
    ngrp = sb_per_w // g

    @functools.partial(
        pl.kernel,
        out_type=[
            jax.ShapeDtypeStruct((n_idx * K // 128, 128), jnp.int32),
            jax.ShapeDtypeStruct((n_idx * K // 128, 128), jnp.int32),
        ],
        mesh=_mesh(),
        compiler_params=_sc_params_tiled(),
        scratch_types=[
            pltpu.VMEM((g, SB), jnp.int32),
            pltpu.VMEM((g, SB), jnp.int32),
            pltpu.VMEM((SB, 128), jnp.int32),
            pltpu.VMEM((SB, 128), jnp.int32),
            pltpu.VMEM((SB, 128), jnp.int32),
            pltpu.VMEM((SB, 128), jnp.int32),
            pltpu.VMEM((g * SB * K // 128, 128), jnp.int32),
            pltpu.VMEM((g * SB * K // 128, 128), jnp.int32),
            pltpu.SemaphoreType.DMA,
            pltpu.SemaphoreType.DMA,
        ],
    )
    def kern(idx_hbm, aep_hbm, arp_hbm, ne2_out, nr2_out,
             idx_v, idx8_v, sta0, sta1, str0, str1, ne2_v, nr2_v,
             sem0, sem1):
        base_sb = _wid() * sb_per_w
        sta = (sta0, sta1)
        stre = (str0, str1)
        sems = (sem0, sem1)
        iota16 = lax.iota(jnp.int32, 16)

        def fire(j, buf):
            return [
                pltpu.async_copy(aep_hbm.at[idx8_v.at[j]], sta[buf], sems[buf]),
                pltpu.async_copy(arp_hbm.at[idx8_v.at[j]], stre[buf], sems[buf]),
            ]

        def extract(j, buf):
            def lg_body(lg, carry):
                col = pl.multiple_of(lg * 16, 16)
                ee = idx_v[j, pl.ds(col, 16)]
                emod = jnp.bitwise_and(ee, 7)
                for m in range(16):
                    mrow = lg * 16 + m
                    prow = j * (SB * K // 128) + 2 * lg + m // 8
                    pcol = (m % 8) * K
                    rowv = jnp.zeros((16,), jnp.int32) + mrow
                    cv = _lane_bcast(emod, m) * 16 + iota16
                    ne2_v[prow, pl.ds(pcol, 16)] = plsc.load_gather(
                        sta[buf], [rowv, cv])
                    nr2_v[prow, pl.ds(pcol, 16)] = plsc.load_gather(
                        stre[buf], [rowv, cv])
                return carry

            lax.fori_loop(0, 8, lg_body, 0)

        def body(i, carry):
            sb0 = base_sb + i * g
            pltpu.sync_copy(idx_hbm.at[pl.ds(sb0, g)], idx_v)
            for j in range(g):
                def pb(lg, c2, _j=j):
                    col = pl.multiple_of(lg * 16, 16)
                    idx8_v[_j, pl.ds(col, 16)] = jnp.right_shift(
                        idx_v[_j, pl.ds(col, 16)], 3)
                    return c2
                lax.fori_loop(0, 8, pb, 0)
            pending = {0: fire(0, 0)}
            for j in range(g):
                buf = j & 1
                if j + 1 < g:
                    pending[1 - buf] = fire(j + 1, 1 - buf)
                for c in pending[buf]:
                    c.wait()
                extract(j, buf)
            pr = g * SB * K // 128
            pltpu.sync_copy(ne2_v, ne2_out.at[pl.ds(sb0 * (SB * K // 128), pr)])
            pltpu.sync_copy(nr2_v, nr2_out.at[pl.ds(sb0 * (SB * K // 128), pr)])
            return carry

        lax.fori_loop(0, ngrp, body, 0, unroll=False)

    return kern


def _make_weighted_agg(n_out, split_w):
    r_per_w = n_out // NW
    bc = min(256, r_per_w)
    nbig = r_per_w // bc
    nsub = bc // SUBC
    sb_per_sub = SUBC * K // SB

    if split_w:
        w_scratch = pltpu.VMEM((2 * bc // K, 128), jnp.float32)
    else:
        w_scratch = pltpu.VMEM((bc * K,), jnp.float32)

    @functools.partial(
        pl.kernel,
        out_type=jax.ShapeDtypeStruct((n_out, 128), jnp.float32),
        mesh=_mesh(),
        compiler_params=_sc_params(),
        scratch_types=[
            pltpu.VMEM((bc * K // SB, SB), jnp.int32),
            pltpu.VMEM((max(bc // SB, 1), SB), jnp.int32),
            w_scratch,
            pltpu.VMEM((SUBC * K, DIM), jnp.float32),
            pltpu.VMEM((SUBC * K, DIM), jnp.float32),
            pltpu.VMEM((SUBC, DIM), jnp.float32),
            pltpu.VMEM((SUBC, DIM), jnp.float32),
            pltpu.VMEM((bc, 128), jnp.float32),
            pltpu.SemaphoreType.DMA,
            pltpu.SemaphoreType.DMA,
        ],
    )
    def kern(nidx_hbm, sidx_hbm, wa_hbm, wb_hbm, table_hbm, out_hbm,
             idx_v, sidx_v, w_v, rows0, rows1, selfs0, selfs1, acc_v,
             sem0, sem1):
        base = _wid() * r_per_w
        rows = (rows0, rows1)
        selfs = (selfs0, selfs1)
        sems = (sem0, sem1)

        def fire(sc, buf):
            cs = [pltpu.async_copy(
                table_hbm.at[idx_v.at[sc * sb_per_sub + j]],
                rows[buf].at[pl.ds(j * SB, SB)], sems[buf])
                for j in range(sb_per_sub)]
            e = sc * SUBC
            sref = sidx_v.at[e // SB, pl.ds(e % SB, SUBC)]
            cs.append(pltpu.async_copy(table_hbm.at[sref], selfs[buf], sems[buf]))
            return cs

        dnums = lax.GatherDimensionNumbers(
            offset_dims=(), collapsed_slice_dims=(0,), start_index_map=(0,))

        def compute(sc, rref, sref):
            def row_body(r, carry2):
                roff = pl.multiple_of(r * K, K)
                arow = sc * SUBC + r
                if split_w:
                    rowsel = arow // K + ((arow // 8) % 2) * (bc // K)
                    wcol = pl.multiple_of((arow % 8) * K, K)
                    wvec = w_v[rowsel, pl.ds(wcol, 16)]
                else:
                    woff = pl.multiple_of(sc * SUBC * K, K) + roff
                    wvec = w_v[pl.ds(woff, 16)]
                acc_a = [sref[r, pl.ds(d * 16, 16)] for d in range(DIM // 16)]
                acc_b = [jnp.zeros((16,), jnp.float32) for _ in range(DIM // 16)]
                for kk in range(K):
                    wb = lax.gather(
                        wvec, jnp.full((16, 1), kk, jnp.int32), dnums, (1,),
                        mode=lax.GatherScatterMode.PROMISE_IN_BOUNDS)
                    tgt = acc_a if kk % 2 == 0 else acc_b
                    for d in range(DIM // 16):
                        tgt[d] = tgt[d] + wb * rref[roff + kk, pl.ds(d * 16, 16)]
                for d in range(DIM // 16):
                    acc_v[arow, pl.ds(d * 16, 16)] = acc_a[d] + acc_b[d]
                return carry2

            lax.fori_loop(0, SUBC, row_body, 0, unroll=4)

        def big_body(bi, carry):
            off = base + bi * bc
            pltpu.sync_copy(nidx_hbm.at[pl.ds(off * K // SB, bc * K // SB)], idx_v)
            pltpu.sync_copy(sidx_hbm.at[pl.ds(off // SB, max(bc // SB, 1))], sidx_v)
            if split_w:
                nb = bc // K
                pltpu.sync_copy(wa_hbm.at[pl.ds(off // K, nb)], w_v.at[pl.ds(0, nb)])
                pltpu.sync_copy(wb_hbm.at[pl.ds(off // K, nb)], w_v.at[pl.ds(nb, nb)])
            else:
                pltpu.sync_copy(wa_hbm.at[pl.ds(off * K, bc * K)], w_v)
            pending = {0: fire(0, 0)}
            for sc in range(nsub):
                buf = sc & 1
                if sc + 1 < nsub:
                    pending[1 - buf] = fire(sc + 1, 1 - buf)
                for c in pending[buf]:
                    c.wait()
                compute(sc, rows[buf], selfs[buf])
            pltpu.sync_copy(acc_v, out_hbm.at[pl.ds(off, bc)])
            return carry

        lax.fori_loop(0, nbig, big_body, 0, unroll=False)

    return kern


def _tc_weights_body(ue_ref, rel_ref, nr1_ref, nr2_ref, w1_ref, w2a_ref, w2b_ref):
    ue = ue_ref[...]
    relm = rel_ref[...]
    urs = lax.dot_general(ue, relm, (((1,), (1,)), ((), ())),
                          preferred_element_type=jnp.float32)
    nr1 = nr1_ref[...]
    nr2 = nr2_ref[...]
    tb = ue.shape[0]
    s1 = jnp.zeros((tb, K), jnp.float32)
    s2 = jnp.zeros((tb, K * K), jnp.float32)
    nrel = relm.shape[0]
    for r in range(nrel):
        c = urs[:, r]
        s1 = s1 + jnp.where(nr1 == r, c[:, None], 0.0)
        s2 = s2 + jnp.where(nr2 == r, c[:, None], 0.0)

    e1 = jnp.exp(s1)
    w1_ref[...] = e1 / jnp.sum(e1, axis=-1, keepdims=True)

    e2 = jnp.exp(s2)
    seg = (lax.broadcasted_iota(jnp.int32, (K * K, K), 0) // K ==
           lax.broadcasted_iota(jnp.int32, (K * K, K), 1)).astype(jnp.float32)
    z = lax.dot_general(e2, seg, (((1,), (0,)), ((), ())),
                        preferred_element_type=jnp.float32)
    zb = lax.dot_general(z, seg, (((1,), (1,)), ((), ())),
                         preferred_element_type=jnp.float32)
    w2 = e2 / zb
    w2a_ref[...] = w2[:, :128]
    w2b_ref[...] = w2[:, 128:]


def _tc_final_body(x1_ref, agg0_ref, w1_ref, ue_ref, wt_ref, b_ref, out_ref):
    tb = w1_ref.shape[0]
    wt = wt_ref[...]
    bias = b_ref[...]
    x1 = x1_ref[...][:, :DIM]
    agg0 = agg0_ref[...][:, :DIM]
    h1 = jax.nn.sigmoid(
        lax.dot_general(x1, wt, (((1,), (0,)), ((), ())),
                        preferred_element_type=jnp.float32) + bias)
    w1 = w1_ref[...]
    aggf = jnp.sum(w1[..., None] * h1.reshape(tb, K, DIM), axis=1)
    h0 = jax.nn.sigmoid(
        lax.dot_general(agg0, wt, (((1,), (0,)), ((), ())),
                        preferred_element_type=jnp.float32) + bias)
    fin = jnp.tanh(
        lax.dot_general(h0 + aggf, wt, (((1,), (0,)), ((), ())),
                        preferred_element_type=jnp.float32) + bias)
    out_ref[...] = jax.nn.sigmoid(jnp.sum(ue_ref[...] * fin, axis=-1))


def kernel(u, v, usr, ent, rel, W, b, adj_ent, adj_rel):
    B = u.shape[0]
    u = u.astype(jnp.int32)
    v = v.astype(jnp.int32)

    usrp = usr.reshape(-1, 128)
    aep = adj_ent.reshape(-1, 128)
    arp = adj_rel.reshape(-1, 128)

    ue, ne1, nr1 = _make_gather_hop0(B)(
        u.reshape(B // SB, SB), v.reshape(B // SB, SB), usrp, aep, arp)
    ne1f = ne1

    ne2, nr2 = _make_gather_hop1(B * K, group=8)(ne1f, aep, arp)

    TB = 256
    grid = (B // TB,)
    w1, w2a, w2b = pl.pallas_call(
        _tc_weights_body,
        grid=grid,
        in_specs=[
            pl.BlockSpec((TB, DIM), lambda i: (i, 0)),
            pl.BlockSpec((rel.shape[0], DIM), lambda i: (0, 0)),
            pl.BlockSpec((TB, K), lambda i: (i, 0)),
            pl.BlockSpec((TB, K * K), lambda i: (i, 0)),
        ],
        out_specs=[
            pl.BlockSpec((TB, K), lambda i: (i, 0)),
            pl.BlockSpec((TB, 128), lambda i: (i, 0)),
            pl.BlockSpec((TB, 128), lambda i: (i, 0)),
        ],
        out_shape=[
            jax.ShapeDtypeStruct((B, K), jnp.float32),
            jax.ShapeDtypeStruct((B, 128), jnp.float32),
            jax.ShapeDtypeStruct((B, 128), jnp.float32),
        ],
    )(ue, rel, nr1.reshape(B, K), nr2.reshape(B, K * K))

    x1 = _make_weighted_agg(B * K, split_w=True)(
        ne2, ne1f, w2a, w2b, ent)
    agg0 = _make_weighted_agg(B, split_w=False)(
        ne1f, v.reshape(B // SB, SB), w1.reshape(B * K), w1.reshape(B * K), ent)

    out = pl.pallas_call(
        _tc_final_body,
        grid=grid,
        in_specs=[
            pl.BlockSpec((TB * K, 128), lambda i: (i, 0)),
            pl.BlockSpec((TB, 128), lambda i: (i, 0)),
            pl.BlockSpec((TB, K), lambda i: (i, 0)),
            pl.BlockSpec((TB, DIM), lambda i: (i, 0)),
            pl.BlockSpec((DIM, DIM), lambda i: (0, 0)),
            pl.BlockSpec((1, DIM), lambda i: (0, 0)),
        ],
        out_specs=pl.BlockSpec((TB,), lambda i: (i,)),
        out_shape=jax.ShapeDtypeStruct((B,), jnp.float32),
    )(x1, agg0, w1, ue, W.T, b.reshape(1, DIM))
    return out


# speedup vs baseline: 1.0380x; 1.0380x over previous
import functools

import jax
import jax.numpy as jnp
from jax import lax
from jax.experimental import pallas as pl
from jax.experimental.pallas import tpu as pltpu
from jax.experimental.pallas import tpu_sc as plsc

NC = 2
NS = 16
NW = NC * NS
SB = 128

DIM = 64
K = 16
SUBC = 32


def _mesh():
    return plsc.VectorSubcoreMesh(core_axis_name="c", subcore_axis_name="s")


def _wid():
    return lax.axis_index("s") * NC + lax.axis_index("c")


def _sc_params():
    return pltpu.CompilerParams(use_tc_tiling_on_sc=False)


def _make_gather_hop0(B):
    assert B == NW * SB

    @functools.partial(
        pl.kernel,
        out_type=[
            jax.ShapeDtypeStruct((B, DIM), jnp.float32),
            jax.ShapeDtypeStruct((B, K), jnp.int32),
            jax.ShapeDtypeStruct((B, K), jnp.int32),
        ],
        mesh=_mesh(),
        compiler_params=_sc_params(),
        scratch_types=[
            pltpu.VMEM((1, SB), jnp.int32),
            pltpu.VMEM((1, SB), jnp.int32),
            pltpu.VMEM((SB, DIM), jnp.float32),
            pltpu.VMEM((SB, K), jnp.int32),
            pltpu.VMEM((SB, K), jnp.int32),
            pltpu.SemaphoreType.DMA,
        ],
    )
    def kern(u_hbm, v_hbm, usr_hbm, ae_hbm, ar_hbm,
             ue_out, ne1_out, nr1_out,
             ui_v, vi_v, ue_v, ne1_v, nr1_v, sem):
        sb0 = _wid()
        pltpu.sync_copy(u_hbm.at[pl.ds(sb0, 1)], ui_v)
        pltpu.sync_copy(v_hbm.at[pl.ds(sb0, 1)], vi_v)
        cs = [
            pltpu.async_copy(usr_hbm.at[ui_v.at[0]], ue_v, sem),
            pltpu.async_copy(ae_hbm.at[vi_v.at[0]], ne1_v, sem),
            pltpu.async_copy(ar_hbm.at[vi_v.at[0]], nr1_v, sem),
        ]
        for c in cs:
            c.wait()
        row0 = sb0 * SB
        pltpu.sync_copy(ue_v, ue_out.at[pl.ds(row0, SB)])
        pltpu.sync_copy(ne1_v, ne1_out.at[pl.ds(row0, SB)])
        pltpu.sync_copy(nr1_v, nr1_out.at[pl.ds(row0, SB)])

    return kern


def _make_gather_hop1(n_idx, group):
    total_sb = n_idx // SB
    sb_per_w = total_sb // NW
    g = min(group, sb_per_w)
    ngrp = sb_per_w // g

    @functools.partial(
        pl.kernel,
        out_type=[
            jax.ShapeDtypeStruct((n_idx, K), jnp.int32),
            jax.ShapeDtypeStruct((n_idx, K), jnp.int32),
        ],
        mesh=_mesh(),
        compiler_params=_sc_params(),
        scratch_types=[
            pltpu.VMEM((g, SB), jnp.int32),
            pltpu.VMEM((g * SB, K), jnp.int32),
            pltpu.VMEM((g * SB, K), jnp.int32),
            pltpu.SemaphoreType.DMA,
        ],
    )
    def kern(idx_hbm, ae_hbm, ar_hbm, ne2_out, nr2_out,
             idx_v, ne2_v, nr2_v, sem):
        base_sb = _wid() * sb_per_w

        def body(i, carry):
            sb0 = base_sb + i * g
            pltpu.sync_copy(idx_hbm.at[pl.ds(sb0, g)], idx_v)
            cs = []
            for j in range(g):
                cs.append(pltpu.async_copy(
                    ae_hbm.at[idx_v.at[j]], ne2_v.at[pl.ds(j * SB, SB)], sem))
                cs.append(pltpu.async_copy(
                    ar_hbm.at[idx_v.at[j]], nr2_v.at[pl.ds(j * SB, SB)], sem))
            for c in cs:
                c.wait()
            row0 = sb0 * SB
            pltpu.sync_copy(ne2_v, ne2_out.at[pl.ds(row0, g * SB)])
            pltpu.sync_copy(nr2_v, nr2_out.at[pl.ds(row0, g * SB)])
            return carry

        lax.fori_loop(0, ngrp, body, 0, unroll=False)

    return kern


def _make_weighted_agg(n_out, split_w):
    r_per_w = n_out // NW
    bc = min(256, r_per_w)
    nbig = r_per_w // bc
    nsub = bc // SUBC
    sb_per_sub = SUBC * K // SB

    if split_w:
        w_scratch = pltpu.VMEM((2 * bc // K, 128), jnp.float32)
    else:
        w_scratch = pltpu.VMEM((bc * K,), jnp.float32)

    @functools.partial(
        pl.kernel,
        out_type=jax.ShapeDtypeStruct((n_out, 128), jnp.float32),
        mesh=_mesh(),
        compiler_params=_sc_params(),
        scratch_types=[
            pltpu.VMEM((bc * K // SB, SB), jnp.int32),
            pltpu.VMEM((max(bc // SB, 1), SB), jnp.int32),
            w_scratch,
            pltpu.VMEM((SUBC * K, DIM), jnp.float32),
            pltpu.VMEM((SUBC * K, DIM), jnp.float32),
            pltpu.VMEM((SUBC, DIM), jnp.float32),
            pltpu.VMEM((SUBC, DIM), jnp.float32),
            pltpu.VMEM((bc, 128), jnp.float32),
            pltpu.SemaphoreType.DMA,
            pltpu.SemaphoreType.DMA,
        ],
    )
    def kern(nidx_hbm, sidx_hbm, wa_hbm, wb_hbm, table_hbm, out_hbm,
             idx_v, sidx_v, w_v, rows0, rows1, selfs0, selfs1, acc_v,
             sem0, sem1):
        base = _wid() * r_per_w
        rows = (rows0, rows1)
        selfs = (selfs0, selfs1)
        sems = (sem0, sem1)

        def fire(sc, buf):
            cs = [pltpu.async_copy(
                table_hbm.at[idx_v.at[sc * sb_per_sub + j]],
                rows[buf].at[pl.ds(j * SB, SB)], sems[buf])
                for j in range(sb_per_sub)]
            e = sc * SUBC
            sref = sidx_v.at[e // SB, pl.ds(e % SB, SUBC)]
            cs.append(pltpu.async_copy(table_hbm.at[sref], selfs[buf], sems[buf]))
            return cs

        dnums = lax.GatherDimensionNumbers(
            offset_dims=(), collapsed_slice_dims=(0,), start_index_map=(0,))

        def compute(sc, rref, sref):
            def row_body(r, carry2):
                roff = pl.multiple_of(r * K, K)
                arow = sc * SUBC + r
                if split_w:
                    rowsel = arow // K + ((arow // 8) % 2) * (bc // K)
                    wcol = pl.multiple_of((arow % 8) * K, K)
                    wvec = w_v[rowsel, pl.ds(wcol, 16)]
                else:
                    woff = pl.multiple_of(sc * SUBC * K, K) + roff
                    wvec = w_v[pl.ds(woff, 16)]
                acc_a = [sref[r, pl.ds(d * 16, 16)] for d in range(DIM // 16)]
                acc_b = [jnp.zeros((16,), jnp.float32) for _ in range(DIM // 16)]
                for kk in range(K):
                    wb = lax.gather(
                        wvec, jnp.full((16, 1), kk, jnp.int32), dnums, (1,),
                        mode=lax.GatherScatterMode.PROMISE_IN_BOUNDS)
                    tgt = acc_a if kk % 2 == 0 else acc_b
                    for d in range(DIM // 16):
                        tgt[d] = tgt[d] + wb * rref[roff + kk, pl.ds(d * 16, 16)]
                for d in range(DIM // 16):
                    acc_v[arow, pl.ds(d * 16, 16)] = acc_a[d] + acc_b[d]
                return carry2

            lax.fori_loop(0, SUBC, row_body, 0, unroll=4)

        def big_body(bi, carry):
            off = base + bi * bc
            pltpu.sync_copy(nidx_hbm.at[pl.ds(off * K // SB, bc * K // SB)], idx_v)
            pltpu.sync_copy(sidx_hbm.at[pl.ds(off // SB, max(bc // SB, 1))], sidx_v)
            if split_w:
                nb = bc // K
                pltpu.sync_copy(wa_hbm.at[pl.ds(off // K, nb)], w_v.at[pl.ds(0, nb)])
                pltpu.sync_copy(wb_hbm.at[pl.ds(off // K, nb)], w_v.at[pl.ds(nb, nb)])
            else:
                pltpu.sync_copy(wa_hbm.at[pl.ds(off * K, bc * K)], w_v)
            pending = {0: fire(0, 0)}
            for sc in range(nsub):
                buf = sc & 1
                if sc + 1 < nsub:
                    pending[1 - buf] = fire(sc + 1, 1 - buf)
                for c in pending[buf]:
                    c.wait()
                compute(sc, rows[buf], selfs[buf])
            pltpu.sync_copy(acc_v, out_hbm.at[pl.ds(off, bc)])
            return carry

        lax.fori_loop(0, nbig, big_body, 0, unroll=False)

    return kern


def _tc_weights_body(ue_ref, rel_ref, nr1_ref, nr2_ref, w1_ref, w2a_ref, w2b_ref):
    ue = ue_ref[...]
    relm = rel_ref[...]
    urs = lax.dot_general(ue, relm, (((1,), (1,)), ((), ())),
                          preferred_element_type=jnp.float32)
    nr1 = nr1_ref[...]
    nr2 = nr2_ref[...]
    tb = ue.shape[0]
    s1 = jnp.zeros((tb, K), jnp.float32)
    s2 = jnp.zeros((tb, K * K), jnp.float32)
    nrel = relm.shape[0]
    for r in range(nrel):
        c = urs[:, r]
        s1 = s1 + jnp.where(nr1 == r, c[:, None], 0.0)
        s2 = s2 + jnp.where(nr2 == r, c[:, None], 0.0)

    e1 = jnp.exp(s1)
    w1_ref[...] = e1 / jnp.sum(e1, axis=-1, keepdims=True)

    e2 = jnp.exp(s2)
    seg = (lax.broadcasted_iota(jnp.int32, (K * K, K), 0) // K ==
           lax.broadcasted_iota(jnp.int32, (K * K, K), 1)).astype(jnp.float32)
    z = lax.dot_general(e2, seg, (((1,), (0,)), ((), ())),
                        preferred_element_type=jnp.float32)
    zb = lax.dot_general(z, seg, (((1,), (1,)), ((), ())),
                         preferred_element_type=jnp.float32)
    w2 = e2 / zb
    w2a_ref[...] = w2[:, :128]
    w2b_ref[...] = w2[:, 128:]


def _tc_final_body(x1_ref, agg0_ref, w1_ref, ue_ref, wt_ref, b_ref, out_ref):
    tb = w1_ref.shape[0]
    wt = wt_ref[...]
    bias = b_ref[...]
    x1 = x1_ref[...][:, :DIM]
    agg0 = agg0_ref[...][:, :DIM]
    h1 = jax.nn.sigmoid(
        lax.dot_general(x1, wt, (((1,), (0,)), ((), ())),
                        preferred_element_type=jnp.float32) + bias)
    w1 = w1_ref[...]
    aggf = jnp.sum(w1[..., None] * h1.reshape(tb, K, DIM), axis=1)
    h0 = jax.nn.sigmoid(
        lax.dot_general(agg0, wt, (((1,), (0,)), ((), ())),
                        preferred_element_type=jnp.float32) + bias)
    fin = jnp.tanh(
        lax.dot_general(h0 + aggf, wt, (((1,), (0,)), ((), ())),
                        preferred_element_type=jnp.float32) + bias)
    out_ref[...] = jax.nn.sigmoid(jnp.sum(ue_ref[...] * fin, axis=-1))


def kernel(u, v, usr, ent, rel, W, b, adj_ent, adj_rel):
    B = u.shape[0]
    u = u.astype(jnp.int32)
    v = v.astype(jnp.int32)

    ue, ne1, nr1 = _make_gather_hop0(B)(
        u.reshape(B // SB, SB), v.reshape(B // SB, SB), usr, adj_ent, adj_rel)

    ne1f = ne1.reshape(B * K // SB, SB)
    ne2, nr2 = _make_gather_hop1(B * K, group=8)(ne1f, adj_ent, adj_rel)

    TB = 256
    grid = (B // TB,)
    w1, w2a, w2b = pl.pallas_call(
        _tc_weights_body,
        grid=grid,
        in_specs=[
            pl.BlockSpec((TB, DIM), lambda i: (i, 0)),
            pl.BlockSpec((rel.shape[0], DIM), lambda i: (0, 0)),
            pl.BlockSpec((TB, K), lambda i: (i, 0)),
            pl.BlockSpec((TB, K * K), lambda i: (i, 0)),
        ],
        out_specs=[
            pl.BlockSpec((TB, K), lambda i: (i, 0)),
            pl.BlockSpec((TB, 128), lambda i: (i, 0)),
            pl.BlockSpec((TB, 128), lambda i: (i, 0)),
        ],
        out_shape=[
            jax.ShapeDtypeStruct((B, K), jnp.float32),
            jax.ShapeDtypeStruct((B, 128), jnp.float32),
            jax.ShapeDtypeStruct((B, 128), jnp.float32),
        ],
    )(ue, rel, nr1, nr2.reshape(B, K * K))

    x1 = _make_weighted_agg(B * K, split_w=True)(
        ne2.reshape(B * K * K // SB, SB), ne1f, w2a, w2b, ent)
    agg0 = _make_weighted_agg(B, split_w=False)(
        ne1f, v.reshape(B // SB, SB), w1.reshape(B * K), w1.reshape(B * K), ent)

    out = pl.pallas_call(
        _tc_final_body,
        grid=grid,
        in_specs=[
            pl.BlockSpec((TB * K, 128), lambda i: (i, 0)),
            pl.BlockSpec((TB, 128), lambda i: (i, 0)),
            pl.BlockSpec((TB, K), lambda i: (i, 0)),
            pl.BlockSpec((TB, DIM), lambda i: (i, 0)),
            pl.BlockSpec((DIM, DIM), lambda i: (0, 0)),
            pl.BlockSpec((1, DIM), lambda i: (0, 0)),
        ],
        out_specs=pl.BlockSpec((TB,), lambda i: (i,)),
        out_shape=jax.ShapeDtypeStruct((B,), jnp.float32),
    )(x1, agg0, w1, ue, W.T, b.reshape(1, DIM))
    return out
